# baseline (speedup 1.0000x reference)
import functools

import jax
import jax.numpy as jnp
from jax import lax
from jax.experimental import pallas as pl
from jax.experimental.pallas import tpu as pltpu, tpu_sc as plsc

VOCAB = 100000
EMB = 64
BATCH = 16384

_info = plsc.get_sparse_core_info()
_NC, _NS, _L = _info.num_cores, _info.num_subcores, _info.num_lanes
_NW = _NC * _NS
_BPW = BATCH // _NW
_CHUNK = 128
_NCHUNK = _BPW // _CHUNK

_mesh = plsc.VectorSubcoreMesh(core_axis_name="c", subcore_axis_name="s")


@functools.partial(
    pl.kernel,
    mesh=_mesh,
    out_type=jax.ShapeDtypeStruct((BATCH, EMB), jnp.float32),
    compiler_params=pltpu.CompilerParams(use_tc_tiling_on_sc=False,
                                         skip_device_barrier=True),
    scratch_types=[
        pltpu.VMEM((_BPW,), jnp.int32),
        pltpu.VMEM((_BPW, EMB), jnp.float32),
        pltpu.SemaphoreType.DMA,
        pltpu.SemaphoreType.DMA,
        pltpu.SemaphoreType.DMA,
        pltpu.SemaphoreType.DMA,
        pltpu.SemaphoreType.DMA,
    ],
)
def _gather_kernel(idx_hbm, table_hbm, out_hbm, idx_v, rows_v,
                   g0, g1, g2, g3, ssem):
    wid = lax.axis_index("s") * _NC + lax.axis_index("c")
    base = wid * _BPW
    gsems = (g0, g1, g2, g3)
    tbl = table_hbm.at[pl.ds(1, VOCAB)]
    pltpu.sync_copy(idx_hbm.at[pl.ds(base, _BPW)], idx_v)
    copies = []
    for j in range(_NCHUNK):
        copies.append(pltpu.async_copy(
            tbl.at[idx_v.at[pl.ds(j * _CHUNK, _CHUNK)]],
            rows_v.at[pl.ds(j * _CHUNK, _CHUNK)],
            gsems[j]))
    stores = []
    for j in range(_NCHUNK):
        copies[j].wait()
        stores.append(pltpu.async_copy(
            rows_v.at[pl.ds(j * _CHUNK, _CHUNK)],
            out_hbm.at[pl.ds(base + j * _CHUNK, _CHUNK)],
            ssem))
    for s in stores:
        s.wait()


def kernel(chain_id, embedding_table):
    return _gather_kernel(chain_id, embedding_table)


# speedup vs baseline: 1.4746x; 1.4746x over previous
import functools

import jax
import jax.numpy as jnp
from jax import lax
from jax.experimental import pallas as pl
from jax.experimental.pallas import tpu as pltpu, tpu_sc as plsc

VOCAB = 100000
EMB = 64
BATCH = 16384

_info = plsc.get_sparse_core_info()
_NC, _NS, _L = _info.num_cores, _info.num_subcores, _info.num_lanes
_NW = _NC * _NS
_BPW = BATCH // _NW

_mesh = plsc.VectorSubcoreMesh(core_axis_name="c", subcore_axis_name="s")


@functools.partial(
    pl.kernel,
    mesh=_mesh,
    out_type=jax.ShapeDtypeStruct((BATCH, EMB), jnp.float32),
    compiler_params=pltpu.CompilerParams(skip_device_barrier=True),
    scratch_types=[
        pltpu.VMEM((_BPW,), jnp.int32),
        pltpu.VMEM((_BPW, EMB), jnp.float32),
        pltpu.SemaphoreType.DMA,
    ],
)
def _gather_kernel(idx_hbm, table_hbm, out_hbm, idx_v, rows_v, sem):
    wid = lax.axis_index("s") * _NC + lax.axis_index("c")
    base = wid * _BPW
    pltpu.sync_copy(idx_hbm.at[pl.ds(base, _BPW)], idx_v)

    def body(g, carry):
        vec = idx_v[pl.ds(g * _L, _L)] + 1
        for j in range(_L):
            pltpu.async_copy(table_hbm.at[pl.ds(vec[j], 1), :],
                             rows_v.at[pl.ds(g * _L + j, 1), :], sem)
        return carry

    lax.fori_loop(0, _BPW // _L, body, 0)
    pltpu.make_async_copy(table_hbm.at[pl.ds(0, _BPW), :], rows_v, sem).wait()
    pltpu.sync_copy(rows_v, out_hbm.at[pl.ds(base, _BPW)])


def kernel(chain_id, embedding_table):
    return _gather_kernel(chain_id, embedding_table)


# speedup vs baseline: 1.7806x; 1.2076x over previous
import functools

import jax
import jax.numpy as jnp
from jax import lax
from jax.experimental import pallas as pl
from jax.experimental.pallas import tpu as pltpu, tpu_sc as plsc

VOCAB = 100000
EMB = 64
BATCH = 16384

_info = plsc.get_sparse_core_info()
_NC, _NS, _L = _info.num_cores, _info.num_subcores, _info.num_lanes
_NW = _NC * _NS
_FPW = EMB // _NW
_OCHUNK = 2048
_NOC = BATCH // _OCHUNK

_mesh = plsc.VectorSubcoreMesh(core_axis_name="c", subcore_axis_name="s")


@functools.partial(
    pl.kernel,
    mesh=_mesh,
    out_type=jax.ShapeDtypeStruct((EMB, BATCH), jnp.float32),
    compiler_params=pltpu.CompilerParams(skip_device_barrier=True,
                                         needs_layout_passes=False),
    scratch_types=[
        pltpu.VMEM((VOCAB + 1,), jnp.float32),
        pltpu.VMEM((BATCH,), jnp.int32),
        pltpu.VMEM((2, _OCHUNK), jnp.float32),
        pltpu.SemaphoreType.DMA,
        pltpu.SemaphoreType.DMA,
    ],
)
def _gather_kernel(idx_hbm, tableT_hbm, outT_hbm, row_v, idx_v, oc_v,
                   isem, osem):
    wid = lax.axis_index("s") * _NC + lax.axis_index("c")
    pltpu.async_copy(idx_hbm, idx_v, isem).wait()
    for f in range(_FPW):
        e = wid * _FPW + f
        pltpu.sync_copy(tableT_hbm.at[e], row_v)
        flushes = {}
        for oc in range(_NOC):
            buf = oc % 2
            if oc >= 2:
                flushes[oc - 2].wait()

            def body(g, carry, oc=oc, buf=buf):
                vec = idx_v[pl.ds(oc * _OCHUNK + g * _L, _L)] + 1
                oc_v[buf, pl.ds(g * _L, _L)] = plsc.load_gather(row_v, [vec])
                return carry

            lax.fori_loop(0, _OCHUNK // _L, body, 0)
            flushes[oc] = pltpu.async_copy(
                oc_v.at[buf],
                outT_hbm.at[e, pl.ds(oc * _OCHUNK, _OCHUNK)],
                osem)
        flushes[_NOC - 2].wait()
        flushes[_NOC - 1].wait()


def kernel(chain_id, embedding_table):
    outT = _gather_kernel(chain_id, embedding_table.T)
    return outT.T


# speedup vs baseline: 2.6793x; 1.5047x over previous
import functools

import jax
import jax.numpy as jnp
from jax import lax
from jax.experimental import pallas as pl
from jax.experimental.pallas import tpu as pltpu, tpu_sc as plsc

VOCAB = 100000
EMB = 64
BATCH = 16384

_info = plsc.get_sparse_core_info()
_NC, _NS, _L = _info.num_cores, _info.num_subcores, _info.num_lanes
_NW = _NC * _NS
_FPW = EMB // _NW
_OCHUNK = 4096
_NOC = BATCH // _OCHUNK

_mesh = plsc.VectorSubcoreMesh(core_axis_name="c", subcore_axis_name="s")


@functools.partial(
    pl.kernel,
    mesh=_mesh,
    out_type=jax.ShapeDtypeStruct((EMB, BATCH), jnp.float32),
    compiler_params=pltpu.CompilerParams(skip_device_barrier=True,
                                         needs_layout_passes=False,
                                         disable_bounds_checks=True),
    scratch_types=[
        pltpu.VMEM((VOCAB + 1,), jnp.float32),
        pltpu.VMEM((BATCH,), jnp.int32),
        pltpu.VMEM((2, _OCHUNK), jnp.float32),
        pltpu.SemaphoreType.DMA,
        pltpu.SemaphoreType.DMA,
        pltpu.SemaphoreType.DMA,
    ],
)
def _gather_kernel(idx_hbm, tableT_hbm, outT_hbm, row_v, idx_v, oc_v,
                   rsem, isem, osem):
    wid = lax.axis_index("s") * _NC + lax.axis_index("c")
    idx_copy = None
    for f in range(_FPW):
        e = wid * _FPW + f
        row_copy = pltpu.async_copy(tableT_hbm.at[e], row_v, rsem)
        if f == 0:
            idx_copy = pltpu.async_copy(idx_hbm, idx_v, isem)
        row_copy.wait()
        if f == 0:
            idx_copy.wait()
        flushes = {}
        for oc in range(_NOC):
            buf = oc % 2
            if oc >= 2:
                flushes[oc - 2].wait()

            @plsc.parallel_loop(0, _OCHUNK // _L, unroll=8)
            def _gather_body(g, oc=oc, buf=buf):
                vec = idx_v[pl.ds(oc * _OCHUNK + g * _L, _L)] + 1
                oc_v[buf, pl.ds(g * _L, _L)] = plsc.load_gather(
                    row_v, [vec])
            flushes[oc] = pltpu.async_copy(
                oc_v.at[buf],
                outT_hbm.at[e, pl.ds(oc * _OCHUNK, _OCHUNK)],
                osem)
        flushes[_NOC - 2].wait()
        flushes[_NOC - 1].wait()


def kernel(chain_id, embedding_table):
    outT = _gather_kernel(chain_id, embedding_table.T)
    return outT.T
